# baseline (speedup 1.0000x reference)
import functools

import jax
import jax.numpy as jnp
from jax import lax
from jax.experimental import pallas as pl
from jax.experimental.pallas import tpu as pltpu
from jax.experimental.pallas import tpu_sc as plsc

DEPTH = 2000
ROWS = 1024
COLS = 20
ROW_WORDS = COLS * DEPTH
ZROWS = 2
NZDMA = 16

_info = plsc.get_sparse_core_info()
_NC, _NS = _info.num_cores, _info.num_subcores
_NW = _NC * _NS
_MPW = ROWS // _NW
_IPW = _MPW * COLS
_NIB = _IPW // 128

_mesh = plsc.VectorSubcoreMesh(core_axis_name="c", subcore_axis_name="s")


@functools.partial(
    pl.kernel,
    mesh=_mesh,
    out_type=jax.ShapeDtypeStruct((ROWS * ROW_WORDS,), jnp.float32),
    scratch_types=[
        pltpu.VMEM((ZROWS * ROW_WORDS,), jnp.float32),
        pltpu.VMEM((_IPW,), jnp.int32),
        pltpu.VMEM((128,), jnp.float32),
        [pltpu.VMEM((128,), jnp.int32) for _ in range(_NIB)],
        pltpu.SemaphoreType.DMA,
        pltpu.SemaphoreType.DMA,
    ],
    compiler_params=pltpu.CompilerParams(needs_layout_passes=False),
)
def _onehot_sc(idx_hbm, zeros_hbm, ones_hbm, out_hbm,
               zbuf, idx_v, ones_v, pos_vs, zsem, ssem):
    wid = lax.axis_index("s") * _NC + lax.axis_index("c")
    pltpu.sync_copy(zeros_hbm, zbuf)
    pltpu.sync_copy(ones_hbm, ones_v)
    pltpu.sync_copy(idx_hbm.at[pl.ds(wid * _IPW, _IPW)], idx_v)

    base = wid * _MPW * ROW_WORDS
    zcopies = [
        pltpu.async_copy(
            zbuf,
            out_hbm.at[pl.ds(base + i * ZROWS * ROW_WORDS, ZROWS * ROW_WORDS)],
            zsem)
        for i in range(NZDMA)
    ]

    lane = lax.iota(jnp.int32, 16)
    for b in range(_NIB):
        for g in range(8):
            f0 = (b * 8 + g) * 16
            pos = (wid * _IPW + f0 + lane) * DEPTH + idx_v[pl.ds(f0, 16)]
            pos_vs[b][pl.ds(g * 16, 16)] = pos

    for c in zcopies:
        c.wait()
    scopies = [
        pltpu.async_copy(ones_v, out_hbm.at[pos_vs[b]], ssem)
        for b in range(_NIB)
    ]
    for c in scopies:
        c.wait()


def kernel(inputs):
    idx = inputs.astype(jnp.int32).reshape(-1)
    zeros = jnp.zeros((ZROWS * ROW_WORDS,), jnp.float32)
    ones = jnp.ones((128,), jnp.float32)
    flat = _onehot_sc(idx, zeros, ones)
    return flat.reshape(ROWS, COLS, DEPTH)


# speedup vs baseline: 1.5172x; 1.5172x over previous
import functools

import jax
import jax.numpy as jnp
from jax import lax
from jax.experimental import pallas as pl
from jax.experimental.pallas import tpu as pltpu
from jax.experimental.pallas import tpu_sc as plsc

DEPTH = 2000
ROWS = 1024
COLS = 20
ZROWS = 2
NCHUNK = 16
CIDX = ZROWS * COLS
CIDX_PAD = 48

_info = plsc.get_sparse_core_info()
_NC, _NS = _info.num_cores, _info.num_subcores
_NW = _NC * _NS
_MPW = ROWS // _NW

_mesh = plsc.VectorSubcoreMesh(core_axis_name="c", subcore_axis_name="s")


@functools.partial(
    pl.kernel,
    mesh=_mesh,
    out_type=jax.ShapeDtypeStruct((ROWS, COLS, DEPTH), jnp.float32),
    scratch_types=[
        pltpu.VMEM((NCHUNK * CIDX_PAD,), jnp.int32),
        pltpu.VMEM((ZROWS, COLS, DEPTH), jnp.float32),
    ],
    compiler_params=pltpu.CompilerParams(needs_layout_passes=False),
)
def _onehot_sc(idx_hbm, zeros_hbm, out_hbm, idx_v, zbuf):
    wid = lax.axis_index("s") * _NC + lax.axis_index("c")
    pltpu.sync_copy(zeros_hbm, zbuf)
    pltpu.sync_copy(idx_hbm.at[pl.ds(wid * NCHUNK * CIDX_PAD, NCHUNK * CIDX_PAD)],
                    idx_v)

    lane = lax.iota(jnp.int32, 16)
    ones_f = jnp.ones((16,), jnp.float32)
    zeros_f = jnp.zeros((16,), jnp.float32)

    for c in range(NCHUNK):
        groups = []
        for g in range(3):
            f = g * 16 + lane
            z = f // COLS
            j = f - z * COLS
            d = idx_v[pl.ds(c * CIDX_PAD + g * 16, 16)]
            mask = None if g < 2 else (lane < CIDX - 32)
            groups.append((z, j, d, mask))
        for z, j, d, mask in groups:
            plsc.store_scatter(zbuf, [z, j, d], ones_f, mask=mask)
        pltpu.sync_copy(zbuf, out_hbm.at[pl.ds(wid * _MPW + c * ZROWS, ZROWS)])
        for z, j, d, mask in groups:
            plsc.store_scatter(zbuf, [z, j, d], zeros_f, mask=mask)


def kernel(inputs):
    idx = inputs.astype(jnp.int32)
    idx_pad = jnp.pad(idx.reshape(ROWS // ZROWS, CIDX),
                      ((0, 0), (0, CIDX_PAD - CIDX)))
    zeros = jnp.zeros((ZROWS, COLS, DEPTH), jnp.float32)
    return _onehot_sc(idx_pad.reshape(-1), zeros)


# speedup vs baseline: 1.5269x; 1.0064x over previous
import functools

import jax
import jax.numpy as jnp
from jax import lax
from jax.experimental import pallas as pl
from jax.experimental.pallas import tpu as pltpu
from jax.experimental.pallas import tpu_sc as plsc

DEPTH = 2000
ROWS = 1024
COLS = 20
ZROWS = 2
NCHUNK = 16
CIDX = ZROWS * COLS
CIDX_PAD = 48

_info = plsc.get_sparse_core_info()
_NC, _NS = _info.num_cores, _info.num_subcores
_NW = _NC * _NS
_MPW = ROWS // _NW

_mesh = plsc.VectorSubcoreMesh(core_axis_name="c", subcore_axis_name="s")


@functools.partial(
    pl.kernel,
    mesh=_mesh,
    out_type=jax.ShapeDtypeStruct((ROWS, COLS, DEPTH), jnp.float32),
    scratch_types=[
        pltpu.VMEM((NCHUNK * CIDX_PAD,), jnp.int32),
        pltpu.VMEM((ZROWS, COLS, DEPTH), jnp.float32),
    ],
    compiler_params=pltpu.CompilerParams(needs_layout_passes=False,
                                         use_tc_tiling_on_sc=True),
)
def _onehot_sc(idx_hbm, zeros_hbm, out_hbm, idx_v, zbuf):
    wid = lax.axis_index("s") * _NC + lax.axis_index("c")
    pltpu.sync_copy(zeros_hbm, zbuf)
    pltpu.sync_copy(idx_hbm.at[pl.ds(wid * NCHUNK * CIDX_PAD, NCHUNK * CIDX_PAD)],
                    idx_v)

    lane = lax.iota(jnp.int32, 16)
    ones_f = jnp.ones((16,), jnp.float32)
    zeros_f = jnp.zeros((16,), jnp.float32)

    for c in range(NCHUNK):
        groups = []
        for g in range(3):
            f = g * 16 + lane
            z = f // COLS
            j = f - z * COLS
            d = idx_v[pl.ds(c * CIDX_PAD + g * 16, 16)]
            mask = None if g < 2 else (lane < CIDX - 32)
            groups.append((z, j, d, mask))
        for z, j, d, mask in groups:
            plsc.store_scatter(zbuf, [z, j, d], ones_f, mask=mask)
        pltpu.sync_copy(zbuf, out_hbm.at[pl.ds(wid * _MPW + c * ZROWS, ZROWS)])
        for z, j, d, mask in groups:
            plsc.store_scatter(zbuf, [z, j, d], zeros_f, mask=mask)


def kernel(inputs):
    idx = inputs.astype(jnp.int32)
    idx_pad = jnp.pad(idx.reshape(ROWS // ZROWS, CIDX),
                      ((0, 0), (0, CIDX_PAD - CIDX)))
    zeros = jnp.zeros((ZROWS, COLS, DEPTH), jnp.float32)
    return _onehot_sc(idx_pad.reshape(-1), zeros)


# speedup vs baseline: 3.4290x; 2.2458x over previous
import functools

import jax
import jax.numpy as jnp
from jax import lax
from jax.experimental import pallas as pl
from jax.experimental.pallas import tpu as pltpu
from jax.experimental.pallas import tpu_sc as plsc

DEPTH = 2000
ROWS = 1024
COLS = 20
DC = 80
NUNITS = COLS * (DEPTH // DC)
RGROUPS = ROWS // 16

_info = plsc.get_sparse_core_info()
_NC, _NS = _info.num_cores, _info.num_subcores
_NW = _NC * _NS
_UPT = -(-NUNITS // _NW)

_mesh = plsc.VectorSubcoreMesh(core_axis_name="c", subcore_axis_name="s")


@functools.partial(
    pl.kernel,
    mesh=_mesh,
    out_type=jax.ShapeDtypeStruct((COLS, DEPTH, ROWS), jnp.float32),
    scratch_types=[
        pltpu.VMEM((ROWS,), jnp.int32),
        pltpu.VMEM((1, DC, ROWS), jnp.float32),
    ],
    compiler_params=pltpu.CompilerParams(needs_layout_passes=False,
                                         use_tc_tiling_on_sc=True),
)
def _onehot_sc(idx_hbm, zeros_hbm, out_hbm, idx_v, buf):
    wid = lax.axis_index("s") * _NC + lax.axis_index("c")
    pltpu.sync_copy(zeros_hbm, buf)

    lane = lax.iota(jnp.int32, 16)
    z16 = jnp.zeros((16,), jnp.int32)
    ones_f = jnp.ones((16,), jnp.float32)
    zeros_f = jnp.zeros((16,), jnp.float32)

    def unit_body(k, carry):
        u = wid + k * _NW

        @pl.when(u < NUNITS)
        def _():
            j = u // (DEPTH // DC)
            d0 = (u % (DEPTH // DC)) * DC
            pltpu.sync_copy(idx_hbm.at[pl.ds(j * ROWS, ROWS)], idx_v)

            def scatter(g, val):
                v = idx_v[pl.ds(g * 16, 16)]
                mask = (v >= d0) & (v < d0 + DC)
                d_id = jnp.clip(v - d0, 0, DC - 1)
                plsc.store_scatter(buf, [z16, d_id, g * 16 + lane], val,
                                   mask=mask)

            def set_body(g, c):
                scatter(g, ones_f)
                return c

            lax.fori_loop(0, RGROUPS, set_body, 0)
            pltpu.sync_copy(buf, out_hbm.at[pl.ds(j, 1), pl.ds(d0, DC)])

            def clr_body(g, c):
                scatter(g, zeros_f)
                return c

            lax.fori_loop(0, RGROUPS, clr_body, 0)

        return carry

    lax.fori_loop(0, _UPT, unit_body, 0)


def kernel(inputs):
    idx_t = inputs.astype(jnp.int32).T.reshape(-1)
    zeros = jnp.zeros((1, DC, ROWS), jnp.float32)
    out = _onehot_sc(idx_t, zeros)
    return out.transpose(2, 0, 1)
